# baseline (speedup 1.0000x reference)
import functools

import jax
import jax.numpy as jnp
import numpy as np
from jax.experimental import pallas as pl
from jax.experimental.pallas import tpu as pltpu


def _intervals_for(L, ratio=0.15, seed=0):
    rng = np.random.default_rng(seed)
    min_win, max_win = 0, int(0.05 * L)
    intervals, durations = [], []
    while sum(durations) < ratio * L:
        random_start = int(rng.integers(0, L - max_win))
        random_end = random_start + int(rng.integers(min_win, max_win))
        random_win = np.arange(random_start, random_end)
        intersections = [len(np.intersect1d(p, random_win)) for p in intervals]
        if sum(intersections) >= random_end - random_start:
            continue
        intervals.append(random_win)
        durations.append(random_end - random_start - sum(intersections))
    return intervals


@functools.lru_cache(maxsize=None)
def _runs_for(L):
    idx = np.arange(L, dtype=np.int64)
    for win in _intervals_for(L):
        src = idx[win[0] - 1] if win[0] else idx[0]
        idx[win] = src
    masked = np.flatnonzero(idx != np.arange(L))
    runs = []
    if masked.size:
        start = prev = int(masked[0])
        val = int(idx[start])
        for i in masked[1:]:
            i = int(i)
            if i == prev + 1 and int(idx[i]) == val:
                prev = i
            else:
                runs.append((start, prev + 1, val))
                start = prev = i
                val = int(idx[i])
        runs.append((start, prev + 1, val))
    return tuple(runs)


_G = 1024


@functools.lru_cache(maxsize=None)
def _dma_plan(L):
    runs = _runs_for(L)
    hulls = []
    groups = []
    mids = []
    for r, (s, e, _) in enumerate(runs):
        h0 = s - s % _G
        h1 = e + (-e) % _G
        hulls.append((h0, h1))
        groups.append((h0, s - h0, min(e - h0, _G), r))
        m_lo = h0 + _G
        m_hi = h1 - _G if e % _G else h1
        if e % _G and h1 - _G > h0:
            groups.append((h1 - _G, 0, e - (h1 - _G), r))
        if m_hi > m_lo:
            mids.append((m_lo, m_hi, r))
    hulls.sort()
    for (a0, a1), (b0, b1) in zip(hulls, hulls[1:]):
        assert a1 < b0, "runs assumed non-adjacent at granule spacing"
    spans, pos = [], 0
    for h0, h1 in hulls:
        if pos < h0:
            spans.append((pos, h0 - pos))
        pos = h1
    if pos < L:
        spans.append((pos, L - pos))
    return spans, groups, mids


def _dma_body(plan, runs, fills_ref, x_hbm, o_hbm, gbuf, sem, fsem, gsem,
              *fbufs):
    spans, groups, mids = plan
    span_copies = []
    for a, n in spans:
        cp = pltpu.make_async_copy(
            x_hbm.at[pl.ds(a, n)], o_hbm.at[pl.ds(a, n)], sem
        )
        cp.start()
        span_copies.append(cp)
    group_in = []
    for k, (g, _, _, _) in enumerate(groups):
        cp = pltpu.make_async_copy(
            x_hbm.at[pl.ds(g, _G)], gbuf.at[k], gsem
        )
        cp.start()
        group_in.append(cp)
    fill_copies = []
    for j, (m_lo, m_hi, r) in enumerate(mids):
        n = m_hi - m_lo
        fbufs[j][...] = jnp.broadcast_to(fills_ref[r], (n,))
        cp = pltpu.make_async_copy(
            fbufs[j].at[pl.ds(0, n)], o_hbm.at[pl.ds(m_lo, n)], fsem
        )
        cp.start()
        fill_copies.append(cp)
    for cp in group_in:
        cp.wait()
    group_out = []
    for k, (g, l0, l1, r) in enumerate(groups):
        gbuf[k, l0:l1] = jnp.broadcast_to(fills_ref[r], (l1 - l0,))
        cp = pltpu.make_async_copy(
            gbuf.at[k], o_hbm.at[pl.ds(g, _G)], gsem
        )
        cp.start()
        group_out.append(cp)
    for cp in span_copies + fill_copies + group_out:
        cp.wait()


def kernel(x):
    L = x.shape[-1]
    runs = _runs_for(L)
    plan = _dma_plan(L)
    spans, groups, mids = plan
    srcs = jnp.asarray([src for (_, _, src) in runs], dtype=jnp.int32)
    fills = x[srcs] if len(runs) else jnp.zeros((1,), x.dtype)
    out = pl.pallas_call(
        functools.partial(_dma_body, plan, runs),
        in_specs=[
            pl.BlockSpec(memory_space=pltpu.VMEM),
            pl.BlockSpec(memory_space=pl.ANY),
        ],
        out_specs=pl.BlockSpec(memory_space=pl.ANY),
        out_shape=jax.ShapeDtypeStruct((L,), x.dtype),
        scratch_shapes=[
            pltpu.VMEM((max(len(groups), 1), _G), jnp.float32),
            pltpu.SemaphoreType.DMA,
            pltpu.SemaphoreType.DMA,
            pltpu.SemaphoreType.DMA,
        ]
        + [pltpu.VMEM((m_hi - m_lo,), jnp.float32) for m_lo, m_hi, _ in mids],
    )(fills, x)
    return out


# speedup vs baseline: 37.8866x; 37.8866x over previous
import functools

import jax
import jax.numpy as jnp
import numpy as np
from jax.experimental import pallas as pl
from jax.experimental.pallas import tpu as pltpu

_BLOCK = 2731 * 1024


def _intervals_for(L, ratio=0.15, seed=0):
    rng = np.random.default_rng(seed)
    min_win, max_win = 0, int(0.05 * L)
    intervals, durations = [], []
    while sum(durations) < ratio * L:
        random_start = int(rng.integers(0, L - max_win))
        random_end = random_start + int(rng.integers(min_win, max_win))
        random_win = np.arange(random_start, random_end)
        intersections = [len(np.intersect1d(p, random_win)) for p in intervals]
        if sum(intersections) >= random_end - random_start:
            continue
        intervals.append(random_win)
        durations.append(random_end - random_start - sum(intersections))
    return intervals


@functools.lru_cache(maxsize=None)
def _runs_for(L):
    idx = np.arange(L, dtype=np.int64)
    for win in _intervals_for(L):
        src = idx[win[0] - 1] if win[0] else idx[0]
        idx[win] = src
    masked = np.flatnonzero(idx != np.arange(L))
    runs = []
    if masked.size:
        start = prev = int(masked[0])
        val = int(idx[start])
        for i in masked[1:]:
            i = int(i)
            if i == prev + 1 and int(idx[i]) == val:
                prev = i
            else:
                runs.append((start, prev + 1, val))
                start = prev = i
                val = int(idx[i])
        runs.append((start, prev + 1, val))
    return tuple(runs)


def _mask_body(block_fills, fills_ref, x_ref, o_ref):
    pid = pl.program_id(0)
    o_ref[...] = x_ref[...]
    for b, fills in block_fills.items():
        @pl.when(pid == b)
        def _fill(fills=fills):
            for ls, le, r in fills:
                o_ref[ls:le] = jnp.broadcast_to(fills_ref[r], (le - ls,))


def kernel(x):
    L = x.shape[-1]
    runs = _runs_for(L)
    grid = pl.cdiv(L, _BLOCK)
    block_fills = {}
    for r, (s, e, _) in enumerate(runs):
        for b in range(s // _BLOCK, (e - 1) // _BLOCK + 1):
            lo, hi = max(s, b * _BLOCK), min(e, (b + 1) * _BLOCK)
            block_fills.setdefault(b, []).append((lo - b * _BLOCK, hi - b * _BLOCK, r))
    srcs = jnp.asarray([src for (_, _, src) in runs], dtype=jnp.int32)
    nf = max(len(runs), 1)
    fills = x[srcs] if len(runs) else jnp.zeros((1,), x.dtype)
    out = pl.pallas_call(
        functools.partial(_mask_body, block_fills),
        grid=(grid,),
        in_specs=[
            pl.BlockSpec((nf,), lambda i: (0,)),
            pl.BlockSpec((_BLOCK,), lambda i: (i,)),
        ],
        out_specs=pl.BlockSpec((_BLOCK,), lambda i: (i,)),
        out_shape=jax.ShapeDtypeStruct((L,), x.dtype),
        compiler_params=pltpu.CompilerParams(
            dimension_semantics=("parallel",),
        ),
    )(fills, x)
    return out
